# Initial kernel scaffold; baseline (speedup 1.0000x reference)
import jax
import jax.numpy as jnp
from jax.experimental import pallas as pl


def kernel(inputs, features, targets):
    raise NotImplementedError("write your pallas kernel here")


# speedup vs baseline: 1.2138x; 1.2138x over previous
import jax
import jax.numpy as jnp
from jax import lax
from jax.experimental import pallas as pl
from jax.experimental.pallas import tpu as pltpu

_TEMP = 0.05
_B = 1024
_D = 64
_N = 100000
_TILE = 2000
_STEPS = _N // _TILE


def _tc_body(x_ref, f_ref, t_ref, out_ref, m_ref, s_ref, tl_ref):
    k = pl.program_id(0)

    @pl.when(k == 0)
    def _init():
        m_ref[...] = jnp.full((_B, 1), -1e30, dtype=jnp.float32)
        s_ref[...] = jnp.zeros((_B, 1), dtype=jnp.float32)
        tl_ref[...] = jnp.zeros((_B, 1), dtype=jnp.float32)

    logits = lax.dot_general(
        x_ref[...], f_ref[...],
        dimension_numbers=(((1,), (1,)), ((), ())),
        preferred_element_type=jnp.float32,
    ) * (1.0 / _TEMP)

    col = k * _TILE + lax.broadcasted_iota(jnp.int32, (_B, _TILE), 1)
    match = col == t_ref[...]
    tl_ref[...] += jnp.sum(jnp.where(match, logits, 0.0), axis=1, keepdims=True)

    tile_max = jnp.max(logits, axis=1, keepdims=True)
    m_old = m_ref[...]
    m_new = jnp.maximum(m_old, tile_max)
    s_ref[...] = (s_ref[...] * jnp.exp(m_old - m_new)
                  + jnp.sum(jnp.exp(logits - m_new), axis=1, keepdims=True))
    m_ref[...] = m_new

    @pl.when(k == _STEPS - 1)
    def _fin():
        lse = m_ref[...] + jnp.log(s_ref[...])
        out_ref[0, 0] = jnp.sum(lse - tl_ref[...]) / jnp.float32(_B)


def kernel(inputs, features, targets):
    loss = pl.pallas_call(
        _tc_body,
        grid=(_STEPS,),
        in_specs=[
            pl.BlockSpec((_B, _D), lambda k: (0, 0)),
            pl.BlockSpec((_TILE, _D), lambda k: (k, 0)),
            pl.BlockSpec((_B, 1), lambda k: (0, 0)),
        ],
        out_specs=pl.BlockSpec((1, 1), lambda k: (0, 0), memory_space=pltpu.SMEM),
        out_shape=jax.ShapeDtypeStruct((1, 1), jnp.float32),
        scratch_shapes=[
            pltpu.VMEM((_B, 1), jnp.float32),
            pltpu.VMEM((_B, 1), jnp.float32),
            pltpu.VMEM((_B, 1), jnp.float32),
        ],
    )(inputs, features, targets.astype(jnp.int32)[:, None])
    return loss[0, 0]


# speedup vs baseline: 1.2195x; 1.0047x over previous
import jax
import jax.numpy as jnp
from jax import lax
from jax.experimental import pallas as pl
from jax.experimental.pallas import tpu as pltpu

_TEMP = 0.05
_B = 1024
_D = 64
_N = 100000
_TILE = 2000
_STEPS = _N // _TILE


def _tc_body(x_ref, f_ref, t_ref, out_ref, m_ref, s_ref, tl_ref):
    k = pl.program_id(0)

    @pl.when(k == 0)
    def _init():
        m_ref[...] = jnp.full((_B, 1), -1e30, dtype=jnp.float32)
        s_ref[...] = jnp.zeros((_B, 1), dtype=jnp.float32)
        tl_ref[...] = jnp.zeros((_B, 1), dtype=jnp.float32)

    logits = lax.dot_general(
        x_ref[...].astype(jnp.bfloat16), f_ref[...].astype(jnp.bfloat16),
        dimension_numbers=(((1,), (1,)), ((), ())),
        preferred_element_type=jnp.float32,
    ) * (1.0 / _TEMP)

    col = k * _TILE + lax.broadcasted_iota(jnp.int32, (_B, _TILE), 1)
    match = col == t_ref[...]
    tl_ref[...] += jnp.sum(jnp.where(match, logits, 0.0), axis=1, keepdims=True)

    tile_max = jnp.max(logits, axis=1, keepdims=True)
    m_old = m_ref[...]
    m_new = jnp.maximum(m_old, tile_max)
    s_ref[...] = (s_ref[...] * jnp.exp(m_old - m_new)
                  + jnp.sum(jnp.exp(logits - m_new), axis=1, keepdims=True))
    m_ref[...] = m_new

    @pl.when(k == _STEPS - 1)
    def _fin():
        lse = m_ref[...] + jnp.log(s_ref[...])
        out_ref[0, 0] = jnp.sum(lse - tl_ref[...]) / jnp.float32(_B)


def kernel(inputs, features, targets):
    loss = pl.pallas_call(
        _tc_body,
        grid=(_STEPS,),
        in_specs=[
            pl.BlockSpec((_B, _D), lambda k: (0, 0)),
            pl.BlockSpec((_TILE, _D), lambda k: (k, 0)),
            pl.BlockSpec((_B, 1), lambda k: (0, 0)),
        ],
        out_specs=pl.BlockSpec((1, 1), lambda k: (0, 0), memory_space=pltpu.SMEM),
        out_shape=jax.ShapeDtypeStruct((1, 1), jnp.float32),
        scratch_shapes=[
            pltpu.VMEM((_B, 1), jnp.float32),
            pltpu.VMEM((_B, 1), jnp.float32),
            pltpu.VMEM((_B, 1), jnp.float32),
        ],
    )(inputs, features, targets.astype(jnp.int32)[:, None])
    return loss[0, 0]
